# baseline (speedup 1.0000x reference)
import jax
import jax.numpy as jnp
from jax import lax
from jax.experimental import pallas as pl
from jax.experimental.pallas import tpu as pltpu


def _add_kernel(x_ref, t_ref, o_ref):
    i = pl.program_id(0)
    bs = x_ref.shape[0]
    seq = t_ref.shape[0]
    start = lax.rem(i * bs, seq)
    o_ref[...] = x_ref[...] + t_ref[pl.ds(start, bs), :]


def kernel(x, table):
    batch, seq, d = x.shape
    n = batch * seq
    xf = x.reshape(n, d)
    bs = 1024
    grid = (n // bs,)
    out = pl.pallas_call(
        _add_kernel,
        grid=grid,
        compiler_params=pltpu.CompilerParams(
            dimension_semantics=("arbitrary",),
        ),
        in_specs=[
            pl.BlockSpec((bs, d), lambda i: (i, 0)),
            pl.BlockSpec((seq, d), lambda i: (0, 0)),
        ],
        out_specs=pl.BlockSpec((bs, d), lambda i: (i, 0)),
        out_shape=jax.ShapeDtypeStruct((n, d), x.dtype),
    )(xf, table)
    return out.reshape(batch, seq, d)


# speedup vs baseline: 1.0666x; 1.0666x over previous
import jax
import jax.numpy as jnp
from jax import lax
from jax.experimental import pallas as pl
from jax.experimental.pallas import tpu as pltpu


def _add_kernel(x_ref, t_ref, o_ref):
    o_ref[...] = x_ref[...] + t_ref[...]


def kernel(x, table):
    batch, seq, d = x.shape
    n = batch * seq
    xf = x.reshape(n, d)
    bs = 2048
    grid = (n // bs,)
    out = pl.pallas_call(
        _add_kernel,
        grid=grid,
        compiler_params=pltpu.CompilerParams(
            dimension_semantics=("arbitrary",),
        ),
        in_specs=[
            pl.BlockSpec((bs, d), lambda i: (i, 0)),
            pl.BlockSpec((seq, d), lambda i: (0, 0)),
        ],
        out_specs=pl.BlockSpec((bs, d), lambda i: (i, 0)),
        out_shape=jax.ShapeDtypeStruct((n, d), x.dtype),
    )(xf, table)
    return out.reshape(batch, seq, d)


# speedup vs baseline: 1.0683x; 1.0016x over previous
import jax
from jax.experimental import pallas as pl
from jax.experimental.pallas import tpu as pltpu


def _add_kernel(x_ref, t_ref, o_ref):
    o_ref[...] = x_ref[...] + t_ref[...]


def kernel(x, table):
    batch, seq, d = x.shape
    n = batch * seq
    xf = x.reshape(n, d)
    bs = seq
    grid = (n // bs,)
    out = pl.pallas_call(
        _add_kernel,
        grid=grid,
        compiler_params=pltpu.CompilerParams(
            dimension_semantics=("arbitrary",),
        ),
        in_specs=[
            pl.BlockSpec((bs, d), lambda i: (i, 0)),
            pl.BlockSpec((seq, d), lambda i: (0, 0)),
        ],
        out_specs=pl.BlockSpec((bs, d), lambda i: (i, 0)),
        out_shape=jax.ShapeDtypeStruct((n, d), x.dtype),
    )(xf, table)
    return out.reshape(batch, seq, d)
